# Initial kernel scaffold; baseline (speedup 1.0000x reference)
import jax
import jax.numpy as jnp
from jax.experimental import pallas as pl


def kernel(x1, x2, edge_index1, edge_index2, params):
    raise NotImplementedError("write your pallas kernel here")


# speedup vs baseline: 4.1643x; 4.1643x over previous
import functools

import jax
import jax.numpy as jnp
from jax import lax
from jax.experimental import pallas as pl
from jax.experimental.pallas import tpu as pltpu
from jax.experimental.pallas import tpu_sc as plsc

N = 10000
E = 160000
H = 128
D_OUT = 64

_B = 2000
_BA = 200
_INV_SCALE = 1.0 / (128.0 ** 0.5)

_NS = 16
_EPT = E // _NS
_CH = 80
_NCHUNK = _EPT // _CH
_NP = 10240
_RPT = _NP // _NS
_ZB = 128
_DEGW = 128


def _dis(deg):
    return jnp.where(deg > 0, lax.rsqrt(jnp.maximum(deg, 1e-12)), 0.0)


def _sigmoid(x):
    return 1.0 / (1.0 + jnp.exp(-x))


def _pre_body(x1, x2, w1, b1, w2, b2, d1, d2, h1_o, h2_o, xs1_o, xs2_o):
    h1 = jnp.maximum(x1[...] @ w1[...] + b1[...], 0.0)
    h2 = jnp.maximum(x2[...] @ w2[...] + b2[...], 0.0)
    h1_o[...] = h1
    h2_o[...] = h2
    xs1_o[...] = _dis(d1[...]) * h1
    xs2_o[...] = _dis(d2[...]) * h2


def _tc_pre(x1, x2, wb1, wb2, deg1, deg2):
    nb = N // _B
    sx = pl.BlockSpec((_B, H), lambda i: (i, 0))
    sw = pl.BlockSpec((H, H), lambda i: (0, 0))
    sb = pl.BlockSpec((1, H), lambda i: (0, 0))
    sd = pl.BlockSpec((_B, 1), lambda i: (i, 0))
    return pl.pallas_call(
        _pre_body,
        grid=(nb,),
        in_specs=[sx, sx, sw, sb, sw, sb, sd, sd],
        out_specs=[sx, sx, sx, sx],
        out_shape=[jax.ShapeDtypeStruct((N, H), jnp.float32)] * 4,
    )(x1, x2, wb1[0], wb1[1].reshape(1, H), wb2[0], wb2[1].reshape(1, H),
      deg1, deg2)


def _convproj_body(h1, g1, d1, h2, g2, d2,
                   w01, w11, bb1, w02, w12, bb2,
                   wq1, bq1, wk1, bk1, wv1, bv1,
                   wq2, bq2, wk2, bk2, wv2, bv2,
                   x1_o, x2_o, q1_o, k1_o, v1_o, q2_o, k2_o, v2_o):
    x1 = jnp.maximum(
        h1[...] @ w01[...] - (_dis(d1[...]) * g1[...]) @ w11[...] + bb1[...],
        0.0)
    x2 = jnp.maximum(
        h2[...] @ w02[...] - (_dis(d2[...]) * g2[...]) @ w12[...] + bb2[...],
        0.0)
    x1_o[...] = x1
    x2_o[...] = x2
    q1_o[...] = x1 @ wq1[...] + bq1[...]
    k1_o[...] = x1 @ wk1[...] + bk1[...]
    v1_o[...] = x1 @ wv1[...] + bv1[...]
    q2_o[...] = x2 @ wq2[...] + bq2[...]
    k2_o[...] = x2 @ wk2[...] + bk2[...]
    v2_o[...] = x2 @ wv2[...] + bv2[...]


def _tc_convproj(h1, g1, deg1, h2, g2, deg2, conv1, conv2, proj):
    nb = N // _B
    sx = pl.BlockSpec((_B, H), lambda i: (i, 0))
    sw = pl.BlockSpec((H, H), lambda i: (0, 0))
    sb = pl.BlockSpec((1, H), lambda i: (0, 0))
    sd = pl.BlockSpec((_B, 1), lambda i: (i, 0))
    w01, w11, bb1 = conv1
    w02, w12, bb2 = conv2
    pj = []
    for wb in proj:
        pj += [wb[0], wb[1].reshape(1, H)]
    return pl.pallas_call(
        _convproj_body,
        grid=(nb,),
        in_specs=[sx, sx, sd, sx, sx, sd]
                 + [sw, sw, sb] * 2 + [sw, sb] * 6,
        out_specs=[sx] * 8,
        out_shape=[jax.ShapeDtypeStruct((N, H), jnp.float32)] * 8,
    )(h1, g1, deg1, h2, g2, deg2,
      w01, w11, bb1.reshape(1, H), w02, w12, bb2.reshape(1, H), *pj)


def _attn_body(q, kf, vf, xres, wo, bo, d, xn_o, xs_o):
    s = lax.dot_general(q[...], kf[...],
                        (((1,), (1,)), ((), ()))) * _INV_SCALE
    m = jnp.max(s, axis=1, keepdims=True)
    p = jnp.exp(s - m)
    denom = jnp.sum(p, axis=1, keepdims=True)
    o = (p @ vf[...]) / denom
    xn = xres[...] + o @ wo[...] + bo[...]
    xn_o[...] = xn
    xs_o[...] = _dis(d[...]) * xn


def _tc_attn(q, k, v, xres, wbo, deg):
    nb = N // _BA
    sx = pl.BlockSpec((_BA, H), lambda i: (i, 0))
    sf = pl.BlockSpec((N, H), lambda i: (0, 0))
    sw = pl.BlockSpec((H, H), lambda i: (0, 0))
    sb = pl.BlockSpec((1, H), lambda i: (0, 0))
    sd = pl.BlockSpec((_BA, 1), lambda i: (i, 0))
    return pl.pallas_call(
        _attn_body,
        grid=(nb,),
        in_specs=[sx, sf, sf, sx, sw, sb, sd],
        out_specs=[sx, sx],
        out_shape=[jax.ShapeDtypeStruct((N, H), jnp.float32)] * 2,
    )(q, k, v, xres, wbo[0], wbo[1].reshape(1, H), deg)


def _final_body(x1, g1, d1, x2, g2, d2,
                w01, w11, bb1, w02, w12, bb2,
                wg1a, wg1b, bg1, wg2, bg2, wc1, bc1, wc2, bc2, out_o):
    x1f = jnp.maximum(
        x1[...] @ w01[...] - (_dis(d1[...]) * g1[...]) @ w11[...] + bb1[...],
        0.0)
    x2f = jnp.maximum(
        x2[...] @ w02[...] - (_dis(d2[...]) * g2[...]) @ w12[...] + bb2[...],
        0.0)
    hg = jnp.maximum(x1f @ wg1a[...] + x2f @ wg1b[...] + bg1[...], 0.0)
    alpha = _sigmoid(hg @ wg2[...] + bg2[...])
    fused = alpha * x1f + (1.0 - alpha) * x2f
    hc = jnp.maximum(fused @ wc1[...] + bc1[...], 0.0)
    out_o[...] = hc @ wc2[...] + bc2[...]


def _tc_final(x1, g1, deg1, x2, g2, deg2, conv1, conv2,
              gate1, gate2, cls1, cls2):
    nb = N // _B
    sx = pl.BlockSpec((_B, H), lambda i: (i, 0))
    sw = pl.BlockSpec((H, H), lambda i: (0, 0))
    sb = pl.BlockSpec((1, H), lambda i: (0, 0))
    sd = pl.BlockSpec((_B, 1), lambda i: (i, 0))
    s1 = pl.BlockSpec((H, 1), lambda i: (0, 0))
    s11 = pl.BlockSpec((1, 1), lambda i: (0, 0))
    sco = pl.BlockSpec((H, D_OUT), lambda i: (0, 0))
    sbo = pl.BlockSpec((1, D_OUT), lambda i: (0, 0))
    so = pl.BlockSpec((_B, D_OUT), lambda i: (i, 0))
    w01, w11, bb1 = conv1
    w02, w12, bb2 = conv2
    wg1, bg1 = gate1
    wg2, bg2 = gate2
    wc1, bc1 = cls1
    wc2, bc2 = cls2
    return pl.pallas_call(
        _final_body,
        grid=(nb,),
        in_specs=[sx, sx, sd, sx, sx, sd,
                  sw, sw, sb, sw, sw, sb,
                  sw, sw, sb, s1, s11, sw, sb, sco, sbo],
        out_specs=so,
        out_shape=jax.ShapeDtypeStruct((N, D_OUT), jnp.float32),
    )(x1, g1, deg1, x2, g2, deg2,
      w01, w11, bb1.reshape(1, H), w02, w12, bb2.reshape(1, H),
      wg1[:H], wg1[H:], bg1.reshape(1, H),
      wg2, bg2.reshape(1, 1), wc1, bc1.reshape(1, H),
      wc2, bc2.reshape(1, D_OUT))


@functools.cache
def _sc_deg_kernel():
    mesh = plsc.VectorSubcoreMesh(core_axis_name="c", subcore_axis_name="s",
                                  num_cores=2, num_subcores=_NS)
    return pl.kernel(
        _sc_deg_body,
        out_type=[jax.ShapeDtypeStruct((_NP, _DEGW), jnp.float32)] * 2,
        mesh=mesh,
        scratch_types=[
            pltpu.VMEM((_CH,), jnp.int32),
            pltpu.VMEM((_CH, _DEGW), jnp.float32),
            pltpu.VMEM((_ZB, _DEGW), jnp.float32),
            pltpu.VMEM_SHARED((_NP, _DEGW), jnp.float32),
        ],
    )


@functools.cache
def _sc_g_kernel():
    mesh = plsc.VectorSubcoreMesh(core_axis_name="c", subcore_axis_name="s",
                                  num_cores=2, num_subcores=_NS)
    return pl.kernel(
        _sc_g_body,
        out_type=[jax.ShapeDtypeStruct((_NP, H), jnp.float32)] * 2,
        mesh=mesh,
        scratch_types=[
            pltpu.VMEM((_CH,), jnp.int32),
            pltpu.VMEM((_CH,), jnp.int32),
            pltpu.VMEM((_CH, H), jnp.float32),
            pltpu.VMEM((_ZB, H), jnp.float32),
            pltpu.VMEM_SHARED((_NP, H), jnp.float32),
            pltpu.SemaphoreType.DMA,
        ],
    )


def _sc_deg(row1, row2):
    return _sc_deg_kernel()(row1, row2)


def _sc_g(xs1, row1, col1, xs2, row2, col2):
    return _sc_g_kernel()(xs1, row1, col1, xs2, row2, col2)


def _sc_deg_body(row1_hbm, row2_hbm, deg1_hbm, deg2_hbm,
            idx_v, ones_v, zbuf_v, acc_sh):
    c = lax.axis_index("c")
    s = lax.axis_index("s")

    def _fill_ones(r, carry):
        for j in range(_DEGW // 16):
            ones_v[r, pl.ds(j * 16, 16)] = jnp.full((16,), 1.0, jnp.float32)
        return carry

    lax.fori_loop(0, _CH, _fill_ones, 0)

    def _fill_zeros(r, carry):
        for j in range(_DEGW // 16):
            zbuf_v[r, pl.ds(j * 16, 16)] = jnp.zeros((16,), jnp.float32)
        return carry

    lax.fori_loop(0, _ZB, _fill_zeros, 0)

    for j in range(_RPT // _ZB):
        pltpu.sync_copy(zbuf_v, acc_sh.at[pl.ds(s * _RPT + j * _ZB, _ZB)])
    plsc.subcore_barrier()

    def _accumulate(row_hbm):
        base0 = s * _EPT

        def body(i, carry):
            pltpu.sync_copy(row_hbm.at[pl.ds(base0 + i * _CH, _CH)], idx_v)
            pltpu.sync_copy(ones_v, acc_sh.at[idx_v], add=True)
            return carry

        lax.fori_loop(0, _NCHUNK, body, 0)

    @pl.when(c == 0)
    def _():
        _accumulate(row1_hbm)

    @pl.when(c == 1)
    def _():
        _accumulate(row2_hbm)

    plsc.subcore_barrier()

    def _copy_out(deg_hbm):
        for j in range(_RPT // _ZB):
            base = s * _RPT + j * _ZB
            pltpu.sync_copy(acc_sh.at[pl.ds(base, _ZB)], zbuf_v)
            pltpu.sync_copy(zbuf_v, deg_hbm.at[pl.ds(base, _ZB)])

    @pl.when(c == 0)
    def _():
        _copy_out(deg1_hbm)

    @pl.when(c == 1)
    def _():
        _copy_out(deg2_hbm)


def _sc_g_body(xs1_hbm, row1_hbm, col1_hbm, xs2_hbm, row2_hbm, col2_hbm,
          g1_hbm, g2_hbm, ridx_v, cidx_v, rows_v, zbuf_v, acc_sh, sem):
    c = lax.axis_index("c")
    s = lax.axis_index("s")

    def _fill_zeros(r, carry):
        for j in range(H // 16):
            zbuf_v[r, pl.ds(j * 16, 16)] = jnp.zeros((16,), jnp.float32)
        return carry

    lax.fori_loop(0, _ZB, _fill_zeros, 0)

    for j in range(_RPT // _ZB):
        pltpu.sync_copy(zbuf_v, acc_sh.at[pl.ds(s * _RPT + j * _ZB, _ZB)])
    plsc.subcore_barrier()

    def _accumulate(xs_hbm, row_hbm, col_hbm):
        base0 = s * _EPT

        def body(i, carry):
            base = base0 + i * _CH
            pltpu.sync_copy(row_hbm.at[pl.ds(base, _CH)], ridx_v)
            pltpu.sync_copy(col_hbm.at[pl.ds(base, _CH)], cidx_v)
            pltpu.async_copy(xs_hbm.at[ridx_v], rows_v, sem).wait()
            pltpu.sync_copy(rows_v, acc_sh.at[cidx_v], add=True)
            return carry

        lax.fori_loop(0, _NCHUNK, body, 0)

    @pl.when(c == 0)
    def _():
        _accumulate(xs1_hbm, row1_hbm, col1_hbm)

    @pl.when(c == 1)
    def _():
        _accumulate(xs2_hbm, row2_hbm, col2_hbm)

    plsc.subcore_barrier()

    def _copy_out(g_hbm):
        for j in range(_RPT // _ZB):
            base = s * _RPT + j * _ZB
            pltpu.sync_copy(acc_sh.at[pl.ds(base, _ZB)], zbuf_v)
            pltpu.sync_copy(zbuf_v, g_hbm.at[pl.ds(base, _ZB)])

    @pl.when(c == 0)
    def _():
        _copy_out(g1_hbm)

    @pl.when(c == 1)
    def _():
        _copy_out(g2_hbm)


def kernel(x1, x2, edge_index1, edge_index2, params):
    p = params
    row1, col1 = edge_index1[0], edge_index1[1]
    row2, col2 = edge_index2[0], edge_index2[1]

    deg1f, deg2f = _sc_deg(row1, row2)
    deg1 = deg1f[:N, 0:1]
    deg2 = deg2f[:N, 0:1]

    h1, h2, xs1, xs2 = _tc_pre(x1, x2, p['lin1_b1'], p['lin1_b2'], deg1, deg2)

    g1, g2 = _sc_g(xs1, row1, col1, xs2, row2, col2)
    proj1 = [p['attn1_' + nm] for nm in ('q1', 'k1', 'v1', 'q2', 'k2', 'v2')]
    x1, x2, q1, k1, v1, q2, k2, v2 = _tc_convproj(
        h1, g1, deg1, h2, g2, deg2, p['conv1_b1'], p['conv1_b2'], proj1)

    x1, xs1 = _tc_attn(q1, k2, v2, x1, p['attn1_o1'], deg1)
    x2, xs2 = _tc_attn(q2, k1, v1, x2, p['attn1_o2'], deg2)

    g1, g2 = _sc_g(xs1, row1, col1, xs2, row2, col2)
    proj2 = [p['attn2_' + nm] for nm in ('q1', 'k1', 'v1', 'q2', 'k2', 'v2')]
    x1, x2, q1, k1, v1, q2, k2, v2 = _tc_convproj(
        x1, g1, deg1, x2, g2, deg2, p['conv2_b1'], p['conv2_b2'], proj2)

    x1, xs1 = _tc_attn(q1, k2, v2, x1, p['attn2_o1'], deg1)
    x2, xs2 = _tc_attn(q2, k1, v1, x2, p['attn2_o2'], deg2)

    g1, g2 = _sc_g(xs1, row1, col1, xs2, row2, col2)
    out = _tc_final(x1, g1, deg1, x2, g2, deg2,
                    p['conv3_b1'], p['conv3_b2'],
                    p['gate1'], p['gate2'], p['cls1'], p['cls2'])
    return out


# speedup vs baseline: 4.9152x; 1.1803x over previous
import functools

import jax
import jax.numpy as jnp
from jax import lax
from jax.experimental import pallas as pl
from jax.experimental.pallas import tpu as pltpu
from jax.experimental.pallas import tpu_sc as plsc

N = 10000
E = 160000
H = 128
D_OUT = 64

_B = 2000
_BA = 200
_INV_SCALE = 1.0 / (128.0 ** 0.5)

_NS = 16
_EPT = E // _NS
_CH = 80
_NCHUNK = _EPT // _CH
_NP = 10240
_RPT = _NP // _NS
_ZB = 128
_DEGW = 128


def _dis(deg):
    return jnp.where(deg > 0, lax.rsqrt(jnp.maximum(deg, 1e-12)), 0.0)


def _sigmoid(x):
    return 1.0 / (1.0 + jnp.exp(-x))


def _pre_body(x1, x2, w1, b1, w2, b2, d1, d2, h1_o, h2_o, xs1_o, xs2_o):
    h1 = jnp.maximum(x1[...] @ w1[...] + b1[...], 0.0)
    h2 = jnp.maximum(x2[...] @ w2[...] + b2[...], 0.0)
    h1_o[...] = h1
    h2_o[...] = h2
    xs1_o[...] = _dis(d1[...]) * h1
    xs2_o[...] = _dis(d2[...]) * h2


def _tc_pre(x1, x2, wb1, wb2, deg1, deg2):
    nb = N // _B
    sx = pl.BlockSpec((_B, H), lambda i: (i, 0))
    sw = pl.BlockSpec((H, H), lambda i: (0, 0))
    sb = pl.BlockSpec((1, H), lambda i: (0, 0))
    sd = pl.BlockSpec((_B, 1), lambda i: (i, 0))
    return pl.pallas_call(
        _pre_body,
        grid=(nb,),
        in_specs=[sx, sx, sw, sb, sw, sb, sd, sd],
        out_specs=[sx, sx, sx, sx],
        out_shape=[jax.ShapeDtypeStruct((N, H), jnp.float32)] * 4,
    )(x1, x2, wb1[0], wb1[1].reshape(1, H), wb2[0], wb2[1].reshape(1, H),
      deg1, deg2)


def _convproj_body(h1, g1, d1, h2, g2, d2,
                   w01, w11, bb1, w02, w12, bb2,
                   wq1, bq1, wk1, bk1, wv1, bv1,
                   wq2, bq2, wk2, bk2, wv2, bv2,
                   x1_o, x2_o, q1_o, k1_o, v1_o, q2_o, k2_o, v2_o):
    x1 = jnp.maximum(
        h1[...] @ w01[...] - (_dis(d1[...]) * g1[...]) @ w11[...] + bb1[...],
        0.0)
    x2 = jnp.maximum(
        h2[...] @ w02[...] - (_dis(d2[...]) * g2[...]) @ w12[...] + bb2[...],
        0.0)
    x1_o[...] = x1
    x2_o[...] = x2
    q1_o[...] = (x1 @ wq1[...] + bq1[...]).astype(jnp.bfloat16)
    k1_o[...] = (x1 @ wk1[...] + bk1[...]).astype(jnp.bfloat16)
    v1_o[...] = (x1 @ wv1[...] + bv1[...]).astype(jnp.bfloat16)
    q2_o[...] = (x2 @ wq2[...] + bq2[...]).astype(jnp.bfloat16)
    k2_o[...] = (x2 @ wk2[...] + bk2[...]).astype(jnp.bfloat16)
    v2_o[...] = (x2 @ wv2[...] + bv2[...]).astype(jnp.bfloat16)


def _tc_convproj(h1, g1, deg1, h2, g2, deg2, conv1, conv2, proj):
    nb = N // _B
    sx = pl.BlockSpec((_B, H), lambda i: (i, 0))
    sw = pl.BlockSpec((H, H), lambda i: (0, 0))
    sb = pl.BlockSpec((1, H), lambda i: (0, 0))
    sd = pl.BlockSpec((_B, 1), lambda i: (i, 0))
    w01, w11, bb1 = conv1
    w02, w12, bb2 = conv2
    pj = []
    for wb in proj:
        pj += [wb[0], wb[1].reshape(1, H)]
    return pl.pallas_call(
        _convproj_body,
        grid=(nb,),
        in_specs=[sx, sx, sd, sx, sx, sd]
                 + [sw, sw, sb] * 2 + [sw, sb] * 6,
        out_specs=[sx] * 8,
        out_shape=[jax.ShapeDtypeStruct((N, H), jnp.float32)] * 2
                  + [jax.ShapeDtypeStruct((N, H), jnp.bfloat16)] * 6,
    )(h1, g1, deg1, h2, g2, deg2,
      w01, w11, bb1.reshape(1, H), w02, w12, bb2.reshape(1, H), *pj)


def _attn_body(q, kf, vf, xres, wo, bo, d, xn_o, xs_o):
    s = lax.dot_general(q[...], kf[...], (((1,), (1,)), ((), ())),
                        preferred_element_type=jnp.float32) * _INV_SCALE
    m = jnp.max(s, axis=1, keepdims=True)
    p = jnp.exp(s - m)
    denom = jnp.sum(p, axis=1, keepdims=True)
    o = lax.dot_general(p.astype(jnp.bfloat16), vf[...],
                        (((1,), (0,)), ((), ())),
                        preferred_element_type=jnp.float32) / denom
    xn = xres[...] + o @ wo[...] + bo[...]
    xn_o[...] = xn
    xs_o[...] = _dis(d[...]) * xn


def _tc_attn(q, k, v, xres, wbo, deg):
    nb = N // _BA
    sx = pl.BlockSpec((_BA, H), lambda i: (i, 0))
    sf = pl.BlockSpec((N, H), lambda i: (0, 0))
    sw = pl.BlockSpec((H, H), lambda i: (0, 0))
    sb = pl.BlockSpec((1, H), lambda i: (0, 0))
    sd = pl.BlockSpec((_BA, 1), lambda i: (i, 0))
    return pl.pallas_call(
        _attn_body,
        grid=(nb,),
        in_specs=[sx, sf, sf, sx, sw, sb, sd],
        out_specs=[sx, sx],
        out_shape=[jax.ShapeDtypeStruct((N, H), jnp.float32)] * 2,
    )(q, k, v, xres, wbo[0], wbo[1].reshape(1, H), deg)


def _final_body(x1, g1, d1, x2, g2, d2,
                w01, w11, bb1, w02, w12, bb2,
                wg1a, wg1b, bg1, wg2, bg2, wc1, bc1, wc2, bc2, out_o):
    x1f = jnp.maximum(
        x1[...] @ w01[...] - (_dis(d1[...]) * g1[...]) @ w11[...] + bb1[...],
        0.0)
    x2f = jnp.maximum(
        x2[...] @ w02[...] - (_dis(d2[...]) * g2[...]) @ w12[...] + bb2[...],
        0.0)
    hg = jnp.maximum(x1f @ wg1a[...] + x2f @ wg1b[...] + bg1[...], 0.0)
    alpha = _sigmoid(hg @ wg2[...] + bg2[...])
    fused = alpha * x1f + (1.0 - alpha) * x2f
    hc = jnp.maximum(fused @ wc1[...] + bc1[...], 0.0)
    out_o[...] = hc @ wc2[...] + bc2[...]


def _tc_final(x1, g1, deg1, x2, g2, deg2, conv1, conv2,
              gate1, gate2, cls1, cls2):
    nb = N // _B
    sx = pl.BlockSpec((_B, H), lambda i: (i, 0))
    sw = pl.BlockSpec((H, H), lambda i: (0, 0))
    sb = pl.BlockSpec((1, H), lambda i: (0, 0))
    sd = pl.BlockSpec((_B, 1), lambda i: (i, 0))
    s1 = pl.BlockSpec((H, 1), lambda i: (0, 0))
    s11 = pl.BlockSpec((1, 1), lambda i: (0, 0))
    sco = pl.BlockSpec((H, D_OUT), lambda i: (0, 0))
    sbo = pl.BlockSpec((1, D_OUT), lambda i: (0, 0))
    so = pl.BlockSpec((_B, D_OUT), lambda i: (i, 0))
    w01, w11, bb1 = conv1
    w02, w12, bb2 = conv2
    wg1, bg1 = gate1
    wg2, bg2 = gate2
    wc1, bc1 = cls1
    wc2, bc2 = cls2
    return pl.pallas_call(
        _final_body,
        grid=(nb,),
        in_specs=[sx, sx, sd, sx, sx, sd,
                  sw, sw, sb, sw, sw, sb,
                  sw, sw, sb, s1, s11, sw, sb, sco, sbo],
        out_specs=so,
        out_shape=jax.ShapeDtypeStruct((N, D_OUT), jnp.float32),
    )(x1, g1, deg1, x2, g2, deg2,
      w01, w11, bb1.reshape(1, H), w02, w12, bb2.reshape(1, H),
      wg1[:H], wg1[H:], bg1.reshape(1, H),
      wg2, bg2.reshape(1, 1), wc1, bc1.reshape(1, H),
      wc2, bc2.reshape(1, D_OUT))


@functools.cache
def _sc_deg_kernel():
    mesh = plsc.VectorSubcoreMesh(core_axis_name="c", subcore_axis_name="s",
                                  num_cores=2, num_subcores=_NS)
    return pl.kernel(
        _sc_deg_body,
        out_type=[jax.ShapeDtypeStruct((_NP, _DEGW), jnp.float32)] * 2,
        mesh=mesh,
        scratch_types=[
            pltpu.VMEM((_CH,), jnp.int32),
            pltpu.VMEM((_CH, _DEGW), jnp.float32),
            pltpu.VMEM((_ZB, _DEGW), jnp.float32),
            pltpu.VMEM_SHARED((_NP, _DEGW), jnp.float32),
        ],
    )


@functools.cache
def _sc_g_kernel():
    mesh = plsc.VectorSubcoreMesh(core_axis_name="c", subcore_axis_name="s",
                                  num_cores=2, num_subcores=_NS)
    return pl.kernel(
        _sc_g_body,
        out_type=[jax.ShapeDtypeStruct((_NP, H), jnp.float32)] * 2,
        mesh=mesh,
        scratch_types=[
            pltpu.VMEM((_CH,), jnp.int32),
            pltpu.VMEM((_CH,), jnp.int32),
            pltpu.VMEM((_CH, H), jnp.float32),
            pltpu.VMEM((_ZB, H), jnp.float32),
            pltpu.VMEM_SHARED((_NP, H), jnp.float32),
            pltpu.SemaphoreType.DMA,
        ],
    )


def _sc_deg(row1, row2):
    return _sc_deg_kernel()(row1, row2)


def _sc_g(xs1, row1, col1, xs2, row2, col2):
    return _sc_g_kernel()(xs1, row1, col1, xs2, row2, col2)


def _sc_deg_body(row1_hbm, row2_hbm, deg1_hbm, deg2_hbm,
            idx_v, ones_v, zbuf_v, acc_sh):
    c = lax.axis_index("c")
    s = lax.axis_index("s")

    def _fill_ones(r, carry):
        for j in range(_DEGW // 16):
            ones_v[r, pl.ds(j * 16, 16)] = jnp.full((16,), 1.0, jnp.float32)
        return carry

    lax.fori_loop(0, _CH, _fill_ones, 0)

    def _fill_zeros(r, carry):
        for j in range(_DEGW // 16):
            zbuf_v[r, pl.ds(j * 16, 16)] = jnp.zeros((16,), jnp.float32)
        return carry

    lax.fori_loop(0, _ZB, _fill_zeros, 0)

    for j in range(_RPT // _ZB):
        pltpu.sync_copy(zbuf_v, acc_sh.at[pl.ds(s * _RPT + j * _ZB, _ZB)])
    plsc.subcore_barrier()

    def _accumulate(row_hbm):
        base0 = s * _EPT

        def body(i, carry):
            pltpu.sync_copy(row_hbm.at[pl.ds(base0 + i * _CH, _CH)], idx_v)
            pltpu.sync_copy(ones_v, acc_sh.at[idx_v], add=True)
            return carry

        lax.fori_loop(0, _NCHUNK, body, 0)

    @pl.when(c == 0)
    def _():
        _accumulate(row1_hbm)

    @pl.when(c == 1)
    def _():
        _accumulate(row2_hbm)

    plsc.subcore_barrier()

    def _copy_out(deg_hbm):
        for j in range(_RPT // _ZB):
            base = s * _RPT + j * _ZB
            pltpu.sync_copy(acc_sh.at[pl.ds(base, _ZB)], zbuf_v)
            pltpu.sync_copy(zbuf_v, deg_hbm.at[pl.ds(base, _ZB)])

    @pl.when(c == 0)
    def _():
        _copy_out(deg1_hbm)

    @pl.when(c == 1)
    def _():
        _copy_out(deg2_hbm)


def _sc_g_body(xs1_hbm, row1_hbm, col1_hbm, xs2_hbm, row2_hbm, col2_hbm,
          g1_hbm, g2_hbm, ridx_v, cidx_v, rows_v, zbuf_v, acc_sh, sem):
    c = lax.axis_index("c")
    s = lax.axis_index("s")

    def _fill_zeros(r, carry):
        for j in range(H // 16):
            zbuf_v[r, pl.ds(j * 16, 16)] = jnp.zeros((16,), jnp.float32)
        return carry

    lax.fori_loop(0, _ZB, _fill_zeros, 0)

    for j in range(_RPT // _ZB):
        pltpu.sync_copy(zbuf_v, acc_sh.at[pl.ds(s * _RPT + j * _ZB, _ZB)])
    plsc.subcore_barrier()

    def _accumulate(xs_hbm, row_hbm, col_hbm):
        base0 = s * _EPT

        def body(i, carry):
            base = base0 + i * _CH
            pltpu.sync_copy(row_hbm.at[pl.ds(base, _CH)], ridx_v)
            pltpu.sync_copy(col_hbm.at[pl.ds(base, _CH)], cidx_v)
            pltpu.async_copy(xs_hbm.at[ridx_v], rows_v, sem).wait()
            pltpu.sync_copy(rows_v, acc_sh.at[cidx_v], add=True)
            return carry

        lax.fori_loop(0, _NCHUNK, body, 0)

    @pl.when(c == 0)
    def _():
        _accumulate(xs1_hbm, row1_hbm, col1_hbm)

    @pl.when(c == 1)
    def _():
        _accumulate(xs2_hbm, row2_hbm, col2_hbm)

    plsc.subcore_barrier()

    def _copy_out(g_hbm):
        for j in range(_RPT // _ZB):
            base = s * _RPT + j * _ZB
            pltpu.sync_copy(acc_sh.at[pl.ds(base, _ZB)], zbuf_v)
            pltpu.sync_copy(zbuf_v, g_hbm.at[pl.ds(base, _ZB)])

    @pl.when(c == 0)
    def _():
        _copy_out(g1_hbm)

    @pl.when(c == 1)
    def _():
        _copy_out(g2_hbm)


def kernel(x1, x2, edge_index1, edge_index2, params):
    p = params
    row1, col1 = edge_index1[0], edge_index1[1]
    row2, col2 = edge_index2[0], edge_index2[1]

    deg1f, deg2f = _sc_deg(row1, row2)
    deg1 = deg1f[:N, 0:1]
    deg2 = deg2f[:N, 0:1]

    h1, h2, xs1, xs2 = _tc_pre(x1, x2, p['lin1_b1'], p['lin1_b2'], deg1, deg2)

    g1, g2 = _sc_g(xs1, row1, col1, xs2, row2, col2)
    proj1 = [p['attn1_' + nm] for nm in ('q1', 'k1', 'v1', 'q2', 'k2', 'v2')]
    x1, x2, q1, k1, v1, q2, k2, v2 = _tc_convproj(
        h1, g1, deg1, h2, g2, deg2, p['conv1_b1'], p['conv1_b2'], proj1)

    x1, xs1 = _tc_attn(q1, k2, v2, x1, p['attn1_o1'], deg1)
    x2, xs2 = _tc_attn(q2, k1, v1, x2, p['attn1_o2'], deg2)

    g1, g2 = _sc_g(xs1, row1, col1, xs2, row2, col2)
    proj2 = [p['attn2_' + nm] for nm in ('q1', 'k1', 'v1', 'q2', 'k2', 'v2')]
    x1, x2, q1, k1, v1, q2, k2, v2 = _tc_convproj(
        x1, g1, deg1, x2, g2, deg2, p['conv2_b1'], p['conv2_b2'], proj2)

    x1, xs1 = _tc_attn(q1, k2, v2, x1, p['attn2_o1'], deg1)
    x2, xs2 = _tc_attn(q2, k1, v1, x2, p['attn2_o2'], deg2)

    g1, g2 = _sc_g(xs1, row1, col1, xs2, row2, col2)
    out = _tc_final(x1, g1, deg1, x2, g2, deg2,
                    p['conv3_b1'], p['conv3_b2'],
                    p['gate1'], p['gate2'], p['cls1'], p['cls2'])
    return out


# speedup vs baseline: 7.2818x; 1.4815x over previous
import functools

import jax
import jax.numpy as jnp
from jax import lax
from jax.experimental import pallas as pl
from jax.experimental.pallas import tpu as pltpu
from jax.experimental.pallas import tpu_sc as plsc

N = 10000
E = 160000
H = 128
D_OUT = 64

_B = 2000
_BA = 200
_INV_SCALE = 1.0 / (128.0 ** 0.5)

_NS = 16
_EPT = E // _NS
_CH = 80
_NCHUNK = _EPT // _CH
_NP = 10240
_RPT = _NP // _NS
_ZB = 32
_DEGW = 128
_D = 4
_GC = _D * _CH
_NG = 31


def _dis(deg):
    return jnp.where(deg > 0, lax.rsqrt(jnp.maximum(deg, 1e-12)), 0.0)


def _sigmoid(x):
    return 1.0 / (1.0 + jnp.exp(-x))


def _pre_body(x1, x2, w1, b1, w2, b2, d1, d2, h1_o, h2_o, xs1_o, xs2_o):
    h1 = jnp.maximum(x1[...] @ w1[...] + b1[...], 0.0)
    h2 = jnp.maximum(x2[...] @ w2[...] + b2[...], 0.0)
    h1_o[...] = h1
    h2_o[...] = h2
    xs1_o[...] = _dis(d1[...]) * h1
    xs2_o[...] = _dis(d2[...]) * h2


def _tc_pre(x1, x2, wb1, wb2, deg1, deg2):
    nb = N // _B
    sx = pl.BlockSpec((_B, H), lambda i: (i, 0))
    sw = pl.BlockSpec((H, H), lambda i: (0, 0))
    sb = pl.BlockSpec((1, H), lambda i: (0, 0))
    sd = pl.BlockSpec((_B, 1), lambda i: (i, 0))
    return pl.pallas_call(
        _pre_body,
        grid=(nb,),
        in_specs=[sx, sx, sw, sb, sw, sb, sd, sd],
        out_specs=[sx, sx, sx, sx],
        out_shape=[jax.ShapeDtypeStruct((N, H), jnp.float32)] * 4,
    )(x1, x2, wb1[0], wb1[1].reshape(1, H), wb2[0], wb2[1].reshape(1, H),
      deg1, deg2)


def _convproj_body(h1, g1, d1, h2, g2, d2,
                   w01, w11, bb1, w02, w12, bb2,
                   wq1, bq1, wk1, bk1, wv1, bv1,
                   wq2, bq2, wk2, bk2, wv2, bv2,
                   x1_o, x2_o, q1_o, k1_o, v1_o, q2_o, k2_o, v2_o):
    x1 = jnp.maximum(
        h1[...] @ w01[...] - (_dis(d1[...]) * g1[...]) @ w11[...] + bb1[...],
        0.0)
    x2 = jnp.maximum(
        h2[...] @ w02[...] - (_dis(d2[...]) * g2[...]) @ w12[...] + bb2[...],
        0.0)
    x1_o[...] = x1
    x2_o[...] = x2
    q1_o[...] = (x1 @ wq1[...] + bq1[...]).astype(jnp.bfloat16)
    k1_o[...] = (x1 @ wk1[...] + bk1[...]).astype(jnp.bfloat16)
    v1_o[...] = (x1 @ wv1[...] + bv1[...]).astype(jnp.bfloat16)
    q2_o[...] = (x2 @ wq2[...] + bq2[...]).astype(jnp.bfloat16)
    k2_o[...] = (x2 @ wk2[...] + bk2[...]).astype(jnp.bfloat16)
    v2_o[...] = (x2 @ wv2[...] + bv2[...]).astype(jnp.bfloat16)


def _tc_convproj(h1, g1, deg1, h2, g2, deg2, conv1, conv2, proj):
    nb = N // _B
    sx = pl.BlockSpec((_B, H), lambda i: (i, 0))
    sw = pl.BlockSpec((H, H), lambda i: (0, 0))
    sb = pl.BlockSpec((1, H), lambda i: (0, 0))
    sd = pl.BlockSpec((_B, 1), lambda i: (i, 0))
    w01, w11, bb1 = conv1
    w02, w12, bb2 = conv2
    pj = []
    for wb in proj:
        pj += [wb[0], wb[1].reshape(1, H)]
    return pl.pallas_call(
        _convproj_body,
        grid=(nb,),
        in_specs=[sx, sx, sd, sx, sx, sd]
                 + [sw, sw, sb] * 2 + [sw, sb] * 6,
        out_specs=[sx] * 8,
        out_shape=[jax.ShapeDtypeStruct((N, H), jnp.float32)] * 2
                  + [jax.ShapeDtypeStruct((N, H), jnp.bfloat16)] * 6,
    )(h1, g1, deg1, h2, g2, deg2,
      w01, w11, bb1.reshape(1, H), w02, w12, bb2.reshape(1, H), *pj)


def _attn_body(q, kf, vf, xres, wo, bo, d, xn_o, xs_o):
    s = lax.dot_general(q[...], kf[...], (((1,), (1,)), ((), ())),
                        preferred_element_type=jnp.float32) * _INV_SCALE
    m = jnp.max(s, axis=1, keepdims=True)
    p = jnp.exp(s - m)
    denom = jnp.sum(p, axis=1, keepdims=True)
    o = lax.dot_general(p.astype(jnp.bfloat16), vf[...],
                        (((1,), (0,)), ((), ())),
                        preferred_element_type=jnp.float32) / denom
    xn = xres[...] + o @ wo[...] + bo[...]
    xn_o[...] = xn
    xs_o[...] = _dis(d[...]) * xn


def _tc_attn(q, k, v, xres, wbo, deg):
    nb = N // _BA
    sx = pl.BlockSpec((_BA, H), lambda i: (i, 0))
    sf = pl.BlockSpec((N, H), lambda i: (0, 0))
    sw = pl.BlockSpec((H, H), lambda i: (0, 0))
    sb = pl.BlockSpec((1, H), lambda i: (0, 0))
    sd = pl.BlockSpec((_BA, 1), lambda i: (i, 0))
    return pl.pallas_call(
        _attn_body,
        grid=(nb,),
        in_specs=[sx, sf, sf, sx, sw, sb, sd],
        out_specs=[sx, sx],
        out_shape=[jax.ShapeDtypeStruct((N, H), jnp.float32)] * 2,
    )(q, k, v, xres, wbo[0], wbo[1].reshape(1, H), deg)


def _final_body(x1, g1, d1, x2, g2, d2,
                w01, w11, bb1, w02, w12, bb2,
                wg1a, wg1b, bg1, wg2, bg2, wc1, bc1, wc2, bc2, out_o):
    x1f = jnp.maximum(
        x1[...] @ w01[...] - (_dis(d1[...]) * g1[...]) @ w11[...] + bb1[...],
        0.0)
    x2f = jnp.maximum(
        x2[...] @ w02[...] - (_dis(d2[...]) * g2[...]) @ w12[...] + bb2[...],
        0.0)
    hg = jnp.maximum(x1f @ wg1a[...] + x2f @ wg1b[...] + bg1[...], 0.0)
    alpha = _sigmoid(hg @ wg2[...] + bg2[...])
    fused = alpha * x1f + (1.0 - alpha) * x2f
    hc = jnp.maximum(fused @ wc1[...] + bc1[...], 0.0)
    out_o[...] = hc @ wc2[...] + bc2[...]


def _tc_final(x1, g1, deg1, x2, g2, deg2, conv1, conv2,
              gate1, gate2, cls1, cls2):
    nb = N // _B
    sx = pl.BlockSpec((_B, H), lambda i: (i, 0))
    sw = pl.BlockSpec((H, H), lambda i: (0, 0))
    sb = pl.BlockSpec((1, H), lambda i: (0, 0))
    sd = pl.BlockSpec((_B, 1), lambda i: (i, 0))
    s1 = pl.BlockSpec((H, 1), lambda i: (0, 0))
    s11 = pl.BlockSpec((1, 1), lambda i: (0, 0))
    sco = pl.BlockSpec((H, D_OUT), lambda i: (0, 0))
    sbo = pl.BlockSpec((1, D_OUT), lambda i: (0, 0))
    so = pl.BlockSpec((_B, D_OUT), lambda i: (i, 0))
    w01, w11, bb1 = conv1
    w02, w12, bb2 = conv2
    wg1, bg1 = gate1
    wg2, bg2 = gate2
    wc1, bc1 = cls1
    wc2, bc2 = cls2
    return pl.pallas_call(
        _final_body,
        grid=(nb,),
        in_specs=[sx, sx, sd, sx, sx, sd,
                  sw, sw, sb, sw, sw, sb,
                  sw, sw, sb, s1, s11, sw, sb, sco, sbo],
        out_specs=so,
        out_shape=jax.ShapeDtypeStruct((N, D_OUT), jnp.float32),
    )(x1, g1, deg1, x2, g2, deg2,
      w01, w11, bb1.reshape(1, H), w02, w12, bb2.reshape(1, H),
      wg1[:H], wg1[H:], bg1.reshape(1, H),
      wg2, bg2.reshape(1, 1), wc1, bc1.reshape(1, H),
      wc2, bc2.reshape(1, D_OUT))


@functools.cache
def _sc_deg_kernel():
    mesh = plsc.VectorSubcoreMesh(core_axis_name="c", subcore_axis_name="s",
                                  num_cores=2, num_subcores=_NS)
    return pl.kernel(
        _sc_deg_body,
        out_type=[jax.ShapeDtypeStruct((_NP, _DEGW), jnp.float32)] * 2,
        mesh=mesh,
        scratch_types=[
            pltpu.VMEM((_EPT,), jnp.int32),
            pltpu.VMEM((_CH, _DEGW), jnp.float32),
            pltpu.VMEM((_ZB, _DEGW), jnp.float32),
            pltpu.VMEM_SHARED((_NP, _DEGW), jnp.float32),
            pltpu.SemaphoreType.DMA,
        ],
    )


@functools.cache
def _sc_g_kernel():
    mesh = plsc.VectorSubcoreMesh(core_axis_name="c", subcore_axis_name="s",
                                  num_cores=2, num_subcores=_NS)
    return pl.kernel(
        _sc_g_body,
        out_type=[jax.ShapeDtypeStruct((_NP, H), jnp.float32)] * 2,
        mesh=mesh,
        scratch_types=[
            pltpu.VMEM((2 * _GC,), jnp.int32),
            pltpu.VMEM((2 * _GC,), jnp.int32),
            pltpu.VMEM((_D, _CH, H), jnp.float32),
            pltpu.VMEM((_ZB, H), jnp.float32),
            pltpu.VMEM_SHARED((_NP, H), jnp.float32),
            pltpu.SemaphoreType.DMA,
            pltpu.SemaphoreType.DMA,
            pltpu.SemaphoreType.DMA,
        ],
    )


def _sc_deg(row1, row2):
    return _sc_deg_kernel()(row1, row2)


def _sc_g(xs1, row1, col1, xs2, row2, col2):
    return _sc_g_kernel()(xs1, row1, col1, xs2, row2, col2)


def _sc_deg_body(row1_hbm, row2_hbm, deg1_hbm, deg2_hbm,
                 idx_v, ones_v, zbuf_v, acc_sh, ssem):
    c = lax.axis_index("c")
    s = lax.axis_index("s")

    def _fill_ones(r, carry):
        for j in range(_DEGW // 16):
            ones_v[r, pl.ds(j * 16, 16)] = jnp.full((16,), 1.0, jnp.float32)
        return carry

    lax.fori_loop(0, _CH, _fill_ones, 0)

    def _fill_zeros(r, carry):
        for j in range(_DEGW // 16):
            zbuf_v[r, pl.ds(j * 16, 16)] = jnp.zeros((16,), jnp.float32)
        return carry

    lax.fori_loop(0, _ZB, _fill_zeros, 0)

    for j in range(_RPT // _ZB):
        pltpu.async_copy(zbuf_v, acc_sh.at[pl.ds(s * _RPT + j * _ZB, _ZB)],
                         ssem)
    for j in range(_RPT // _ZB):
        pltpu.make_async_copy(zbuf_v, acc_sh.at[pl.ds(s * _RPT, _ZB)],
                              ssem).wait()
    plsc.subcore_barrier()

    def _accumulate(row_hbm):
        base0 = s * _EPT
        pltpu.sync_copy(row_hbm.at[pl.ds(base0, _EPT)], idx_v)

        def _fire(i):
            pltpu.async_copy(
                ones_v, acc_sh.at[idx_v.at[pl.ds(i * _CH, _CH)]], ssem,
                add=True)

        def _wait():
            pltpu.make_async_copy(
                ones_v, acc_sh.at[idx_v.at[pl.ds(0, _CH)]], ssem).wait()

        for i in range(10):
            _fire(i)

        @pl.loop(10, _NCHUNK)
        def _slide(i):
            _wait()
            _fire(i)

        for i in range(10):
            _wait()

    @pl.when(c == 0)
    def _():
        _accumulate(row1_hbm)

    @pl.when(c == 1)
    def _():
        _accumulate(row2_hbm)

    plsc.subcore_barrier()

    def _copy_out(deg_hbm):
        pltpu.sync_copy(acc_sh.at[pl.ds(s * _RPT, _RPT)],
                        deg_hbm.at[pl.ds(s * _RPT, _RPT)])

    @pl.when(c == 0)
    def _():
        _copy_out(deg1_hbm)

    @pl.when(c == 1)
    def _():
        _copy_out(deg2_hbm)


def _sc_g_body(xs1_hbm, row1_hbm, col1_hbm, xs2_hbm, row2_hbm, col2_hbm,
               g1_hbm, g2_hbm, ri_v, ci_v, rows_v, zbuf_v, acc_sh,
               isem, gsem, ssem):
    c = lax.axis_index("c")
    s = lax.axis_index("s")

    def _fill_zeros(r, carry):
        for j in range(H // 16):
            zbuf_v[r, pl.ds(j * 16, 16)] = jnp.zeros((16,), jnp.float32)
        return carry

    lax.fori_loop(0, _ZB, _fill_zeros, 0)

    for j in range(_RPT // _ZB):
        pltpu.async_copy(zbuf_v, acc_sh.at[pl.ds(s * _RPT + j * _ZB, _ZB)],
                         ssem)
    for j in range(_RPT // _ZB):
        pltpu.make_async_copy(zbuf_v, acc_sh.at[pl.ds(s * _RPT, _ZB)],
                              ssem).wait()
    plsc.subcore_barrier()

    def _accumulate(xs_hbm, row_hbm, col_hbm):
        base0 = s * _EPT

        def fire_idx(j, p):
            pltpu.async_copy(row_hbm.at[pl.ds(base0 + j * _GC, _GC)],
                             ri_v.at[pl.ds(p * _GC, _GC)], isem)
            pltpu.async_copy(col_hbm.at[pl.ds(base0 + j * _GC, _GC)],
                             ci_v.at[pl.ds(p * _GC, _GC)], isem)

        def wait_idx():
            pltpu.make_async_copy(row_hbm.at[pl.ds(base0, _GC)],
                                  ri_v.at[pl.ds(0, _GC)], isem).wait()
            pltpu.make_async_copy(col_hbm.at[pl.ds(base0, _GC)],
                                  ci_v.at[pl.ds(0, _GC)], isem).wait()

        def fire_gather(p, b):
            pltpu.async_copy(
                xs_hbm.at[ri_v.at[pl.ds(p * _GC + b * _CH, _CH)]],
                rows_v.at[b], gsem)

        def wait_gather(b):
            pltpu.make_async_copy(
                xs_hbm.at[ri_v.at[pl.ds(0, _CH)]], rows_v.at[b],
                gsem).wait()

        def fire_scatter(p, b):
            pltpu.async_copy(
                rows_v.at[b],
                acc_sh.at[ci_v.at[pl.ds(p * _GC + b * _CH, _CH)]],
                ssem, add=True)

        def wait_scatter(b):
            pltpu.make_async_copy(
                rows_v.at[b], acc_sh.at[ci_v.at[pl.ds(0, _CH)]],
                ssem).wait()

        fire_idx(0, 0)
        wait_idx()
        fire_idx(1, 1)
        for b in range(_D):
            fire_gather(0, b)

        @pl.loop(1, _NG - 1)
        def _mid(j):
            pm = lax.rem(j - 1, 2)
            pj = lax.rem(j, 2)
            wait_idx()
            for b in range(_D):
                wait_gather(b)
                fire_scatter(pm, b)
            for b in range(_D):
                wait_scatter(b)
                fire_gather(pj, b)
            fire_idx(j + 1, pm)

        pm = (_NG - 2) % 2
        pj = (_NG - 1) % 2
        wait_idx()
        for b in range(_D):
            wait_gather(b)
            fire_scatter(pm, b)
        for b in range(_D):
            wait_scatter(b)
            fire_gather(pj, b)
        for b in range(_D):
            wait_gather(b)
            fire_scatter(pj, b)
        for b in range(_D):
            wait_scatter(b)

        tb = base0 + _NG * _GC
        pltpu.sync_copy(row_hbm.at[pl.ds(tb, _CH)], ri_v.at[pl.ds(0, _CH)])
        pltpu.sync_copy(col_hbm.at[pl.ds(tb, _CH)], ci_v.at[pl.ds(0, _CH)])
        pltpu.async_copy(xs_hbm.at[ri_v.at[pl.ds(0, _CH)]], rows_v.at[0],
                         gsem).wait()
        pltpu.sync_copy(rows_v.at[0], acc_sh.at[ci_v.at[pl.ds(0, _CH)]],
                        add=True)

    @pl.when(c == 0)
    def _():
        _accumulate(xs1_hbm, row1_hbm, col1_hbm)

    @pl.when(c == 1)
    def _():
        _accumulate(xs2_hbm, row2_hbm, col2_hbm)

    plsc.subcore_barrier()

    def _copy_out(g_hbm):
        pltpu.sync_copy(acc_sh.at[pl.ds(s * _RPT, _RPT)],
                        g_hbm.at[pl.ds(s * _RPT, _RPT)])

    @pl.when(c == 0)
    def _():
        _copy_out(g1_hbm)

    @pl.when(c == 1)
    def _():
        _copy_out(g2_hbm)


def kernel(x1, x2, edge_index1, edge_index2, params):
    p = params
    row1, col1 = edge_index1[0], edge_index1[1]
    row2, col2 = edge_index2[0], edge_index2[1]

    deg1f, deg2f = _sc_deg(row1, row2)
    deg1 = deg1f[:N, 0:1]
    deg2 = deg2f[:N, 0:1]

    h1, h2, xs1, xs2 = _tc_pre(x1, x2, p['lin1_b1'], p['lin1_b2'], deg1, deg2)

    g1, g2 = _sc_g(xs1, row1, col1, xs2, row2, col2)
    proj1 = [p['attn1_' + nm] for nm in ('q1', 'k1', 'v1', 'q2', 'k2', 'v2')]
    x1, x2, q1, k1, v1, q2, k2, v2 = _tc_convproj(
        h1, g1, deg1, h2, g2, deg2, p['conv1_b1'], p['conv1_b2'], proj1)

    x1, xs1 = _tc_attn(q1, k2, v2, x1, p['attn1_o1'], deg1)
    x2, xs2 = _tc_attn(q2, k1, v1, x2, p['attn1_o2'], deg2)

    g1, g2 = _sc_g(xs1, row1, col1, xs2, row2, col2)
    proj2 = [p['attn2_' + nm] for nm in ('q1', 'k1', 'v1', 'q2', 'k2', 'v2')]
    x1, x2, q1, k1, v1, q2, k2, v2 = _tc_convproj(
        x1, g1, deg1, x2, g2, deg2, p['conv2_b1'], p['conv2_b2'], proj2)

    x1, xs1 = _tc_attn(q1, k2, v2, x1, p['attn2_o1'], deg1)
    x2, xs2 = _tc_attn(q2, k1, v1, x2, p['attn2_o2'], deg2)

    g1, g2 = _sc_g(xs1, row1, col1, xs2, row2, col2)
    out = _tc_final(x1, g1, deg1, x2, g2, deg2,
                    p['conv3_b1'], p['conv3_b2'],
                    p['gate1'], p['gate2'], p['cls1'], p['cls2'])
    return out
